# baseline (speedup 1.0000x reference)
import functools

import jax
import jax.numpy as jnp
from jax import lax
from jax.experimental import pallas as pl
from jax.experimental.pallas import tpu as pltpu
from jax.experimental.pallas import tpu_sc as plsc

NC = 2
NS = 16
L = 16
NW = NC * NS

B = 16384
D = 32
V = 1000000
BPW = B // NW
CHUNK = 128
NCHUNK = BPW // CHUNK

_mesh = plsc.VectorSubcoreMesh(core_axis_name="c", subcore_axis_name="s")


@functools.partial(
    pl.kernel,
    out_type=jax.ShapeDtypeStruct((B,), jnp.float32),
    mesh=_mesh,
    compiler_params=pltpu.CompilerParams(needs_layout_passes=False,
                                         use_tc_tiling_on_sc=True),
    scratch_types=[
        pltpu.VMEM((NCHUNK, CHUNK), jnp.int32),
        pltpu.VMEM((NCHUNK, CHUNK), jnp.int32),
        pltpu.VMEM_SHARED((V,), jnp.float32),
        pltpu.VMEM_SHARED((V,), jnp.float32),
        pltpu.VMEM((BPW,), jnp.float32),
        pltpu.VMEM((BPW,), jnp.float32),
        pltpu.VMEM((BPW,), jnp.float32),
        pltpu.SemaphoreType.DMA,
    ],
)
def _mf_sc_kernel(uidx_hbm, iidx_hbm, utabT_hbm, itabT_hbm, out_hbm,
                  uidx_v, iidx_v, upl_s, ipl_s, ug_v, vg_v, acc_v, sem):
    sid = lax.axis_index("s")
    cid = lax.axis_index("c")
    wid = sid * NC + cid
    base = wid * BPW

    for c in range(NCHUNK):
        pltpu.sync_copy(uidx_hbm.at[pl.ds(base + c * CHUNK, CHUNK)], uidx_v.at[c])
        pltpu.sync_copy(iidx_hbm.at[pl.ds(base + c * CHUNK, CHUNK)], iidx_v.at[c])

    zeros = jnp.zeros((L,), jnp.float32)

    def init(j, c):
        acc_v[pl.ds(j * L, L)] = zeros
        return c

    lax.fori_loop(0, BPW // L, init, 0)

    def dbody(d, carry):
        @pl.when(sid == 0)
        def _():
            cp1 = pltpu.async_copy(utabT_hbm.at[d], upl_s, sem)
            cp2 = pltpu.async_copy(itabT_hbm.at[d], ipl_s, sem)
            cp1.wait()
            cp2.wait()
        plsc.subcore_barrier()

        for c in range(NCHUNK):
            pltpu.async_copy(upl_s.at[uidx_v.at[c]],
                             ug_v.at[pl.ds(c * CHUNK, CHUNK)], sem).wait()
            pltpu.async_copy(ipl_s.at[iidx_v.at[c]],
                             vg_v.at[pl.ds(c * CHUNK, CHUNK)], sem).wait()

        def jbody(j, c):
            sl = pl.ds(j * L, L)
            acc_v[sl] = acc_v[sl] + ug_v[sl] * vg_v[sl]
            return c

        lax.fori_loop(0, BPW // L, jbody, 0)
        plsc.subcore_barrier()
        return carry

    lax.fori_loop(0, D, dbody, 0)
    pltpu.sync_copy(acc_v, out_hbm.at[pl.ds(base, BPW)])


def kernel(user_idx, item_idx, user_table, item_table):
    return _mf_sc_kernel(user_idx.astype(jnp.int32), item_idx.astype(jnp.int32),
                         user_table.T, item_table.T)


# speedup vs baseline: 1.7670x; 1.7670x over previous
import functools

import jax
import jax.numpy as jnp
from jax import lax
from jax.experimental import pallas as pl
from jax.experimental.pallas import tpu as pltpu
from jax.experimental.pallas import tpu_sc as plsc

NC = 2
NS = 16
L = 16
NW = NC * NS

B = 16384
D = 32
V = 1000000
BPW = B // NW
E = 16
NE = BPW // E

_mesh = plsc.VectorSubcoreMesh(core_axis_name="c", subcore_axis_name="s")


@functools.partial(
    pl.kernel,
    out_type=jax.ShapeDtypeStruct((B,), jnp.float32),
    mesh=_mesh,
    compiler_params=pltpu.CompilerParams(needs_layout_passes=False,
                                         use_tc_tiling_on_sc=True),
    scratch_types=[
        pltpu.VMEM((BPW + L,), jnp.int32),
        pltpu.VMEM((BPW + L,), jnp.int32),
        pltpu.VMEM((E, 4, 8, 128), jnp.float32),
        pltpu.VMEM((D, L), jnp.float32),
        pltpu.VMEM((BPW,), jnp.float32),
        pltpu.SemaphoreType.DMA,
    ],
)
def _mf_sc_kernel(uidx_hbm, iidx_hbm, utab3_hbm, itab3_hbm, out_hbm,
                  uidx_v, iidx_v, blk_v, ucol_v, out_v, sem):
    sid = lax.axis_index("s")
    cid = lax.axis_index("c")
    wid = sid * NC + cid
    base = wid * BPW

    pltpu.sync_copy(uidx_hbm.at[pl.ds(base, BPW)], uidx_v.at[pl.ds(0, BPW)])
    pltpu.sync_copy(iidx_hbm.at[pl.ds(base, BPW)], iidx_v.at[pl.ds(0, BPW)])

    lane = lax.iota(jnp.int32, L)

    def fetch_blocks(tab_hbm, idx_ref, e0):
        def issue(e, c):
            vec = idx_ref[pl.ds(e0 + e, L)]
            idx = vec[0]
            o0 = pl.multiple_of(lax.bitwise_and(idx, jnp.int32(~127)), 128)
            pltpu.async_copy(tab_hbm.at[:, :, pl.ds(o0, 128)], blk_v.at[e], sem)
            return c
        lax.fori_loop(0, E, issue, 0)
        drain = pltpu.make_async_copy(tab_hbm.at[:, :, pl.ds(0, 128)],
                                      blk_v.at[0], sem)
        for _ in range(E):
            drain.wait()

    def chunk_body(ch, carry):
        e0 = ch * E
        fetch_blocks(utab3_hbm, uidx_v, e0)
        up = lax.bitwise_and(uidx_v[pl.ds(e0, L)], jnp.int32(127))
        for j in range(D):
            fb = jnp.full((L,), j // 8, jnp.int32)
            fr = jnp.full((L,), j % 8, jnp.int32)
            ucol_v[j, :] = plsc.load_gather(blk_v, [lane, fb, fr, up])
        fetch_blocks(itab3_hbm, iidx_v, e0)
        ip = lax.bitwise_and(iidx_v[pl.ds(e0, L)], jnp.int32(127))
        acc = jnp.zeros((L,), jnp.float32)
        for j in range(D):
            fb = jnp.full((L,), j // 8, jnp.int32)
            fr = jnp.full((L,), j % 8, jnp.int32)
            vj = plsc.load_gather(blk_v, [lane, fb, fr, ip])
            acc = acc + ucol_v[j, :] * vj
        out_v[pl.ds(e0, L)] = acc
        return carry

    lax.fori_loop(0, NE, chunk_body, 0)
    pltpu.sync_copy(out_v, out_hbm.at[pl.ds(base, BPW)])


def kernel(user_idx, item_idx, user_table, item_table):
    ut3 = user_table.T.reshape(4, 8, V)
    it3 = item_table.T.reshape(4, 8, V)
    return _mf_sc_kernel(user_idx.astype(jnp.int32), item_idx.astype(jnp.int32),
                         ut3, it3)
